# baseline (speedup 1.0000x reference)
import functools

import jax
import jax.numpy as jnp
from jax import lax
from jax.experimental import pallas as pl
from jax.experimental.pallas import tpu as pltpu
from jax.experimental.pallas import tpu_sc as plsc

BATCH = 16384
VOCAB = 1000000
EMB_DIM = 2
N_CLASSES = 100

_NC = 2
_NS = 16
_NW = _NC * _NS
_PW = BATCH // _NW
_NG = _PW // 16


def _sc_gather(table, idx):
    mesh = plsc.VectorSubcoreMesh(core_axis_name="c", subcore_axis_name="s")

    @functools.partial(
        pl.kernel,
        out_type=jax.ShapeDtypeStruct((EMB_DIM, BATCH), jnp.float32),
        mesh=mesh,
        scratch_types=[
            pltpu.VMEM((_PW,), jnp.int32),
            pltpu.VMEM((_PW, EMB_DIM), jnp.float32),
            pltpu.VMEM((_PW,), jnp.float32),
            pltpu.VMEM((_PW,), jnp.float32),
            pltpu.SemaphoreType.DMA,
        ],
        compiler_params=pltpu.CompilerParams(
            use_tc_tiling_on_sc=True, needs_layout_passes=False
        ),
    )
    def gather_kernel(tbl, idx_h, out_h, idx_v, buf_v, e0_v, e1_v, sem):
        wid = lax.axis_index("s") * _NC + lax.axis_index("c")
        base = wid * _PW
        pltpu.sync_copy(idx_h.at[wid], idx_v)
        lanes = lax.iota(jnp.int32, 16)

        def fire_group(m):
            v = idx_v[pl.ds(m * 16, 16)]
            for t in range(16):
                i0 = lax.reduce_sum(jnp.where(lanes == t, v, 0), axes=(0,))
                pltpu.async_copy(
                    tbl.at[pl.ds(i0, 1)], buf_v.at[pl.ds(m * 16 + t, 1)], sem
                )

        def drain_group(m):
            for t in range(16):
                pltpu.make_async_copy(
                    tbl.at[pl.ds(0, 1)], buf_v.at[pl.ds(m * 16 + t, 1)], sem
                ).wait()

        def body(m, carry):
            fire_group(m)

            @pl.when(m > 0)
            def _():
                drain_group(m - 1)

            return carry

        lax.fori_loop(0, _NG, body, 0)
        drain_group(_NG - 1)

        zeros = jnp.zeros((16,), jnp.int32)
        ones = zeros + 1
        for m in range(_NG):
            rid = lanes + (m * 16)
            e0_v[pl.ds(m * 16, 16)] = plsc.load_gather(buf_v, [rid, zeros])
            e1_v[pl.ds(m * 16, 16)] = plsc.load_gather(buf_v, [rid, ones])
        pltpu.sync_copy(e0_v, out_h.at[0, pl.ds(base, _PW)])
        pltpu.sync_copy(e1_v, out_h.at[1, pl.ds(base, _PW)])

    return gather_kernel(table, idx)


def _tc_dense_softmax(eT, wt, b2):
    rows = 2048
    grid = BATCH // rows

    def body(et_ref, wt_ref, b_ref, out_ref):
        logits = lax.dot_general(
            et_ref[...], wt_ref[...],
            (((0,), (0,)), ((), ())),
            preferred_element_type=jnp.float32,
        ) + b_ref[...]
        m = jnp.max(logits, axis=1, keepdims=True)
        p = jnp.exp(logits - m)
        out_ref[...] = p / jnp.sum(p, axis=1, keepdims=True)

    return pl.pallas_call(
        body,
        grid=(grid,),
        in_specs=[
            pl.BlockSpec((EMB_DIM, rows), lambda i: (0, i)),
            pl.BlockSpec((EMB_DIM, N_CLASSES), lambda i: (0, 0)),
            pl.BlockSpec((1, N_CLASSES), lambda i: (0, 0)),
        ],
        out_specs=pl.BlockSpec((rows, N_CLASSES), lambda i: (i, 0)),
        out_shape=jax.ShapeDtypeStruct((BATCH, N_CLASSES), jnp.float32),
    )(eT, wt, b2)


def _tc_fused(idx, emb, wt, b2):
    rows = 2048
    grid = BATCH // rows

    def body(xs_ref, emb_ref, wt_ref, b_ref, out_ref, buf, sem):
        step = pl.program_id(0)
        base = step * rows

        def issue(k, carry):
            ix = xs_ref[base + k]
            pltpu.make_async_copy(
                emb_ref.at[pl.ds(ix, 1)], buf.at[pl.ds(k, 1)], sem
            ).start()
            return carry

        lax.fori_loop(0, rows, issue, 0)

        def drain(k, carry):
            pltpu.make_async_copy(
                emb_ref.at[pl.ds(0, 1)], buf.at[pl.ds(k, 1)], sem
            ).wait()
            return carry

        lax.fori_loop(0, rows, drain, 0)

        logits = jnp.dot(
            buf[...], wt_ref[...], preferred_element_type=jnp.float32
        ) + b_ref[...]
        m = jnp.max(logits, axis=1, keepdims=True)
        p = jnp.exp(logits - m)
        out_ref[...] = p / jnp.sum(p, axis=1, keepdims=True)

    grid_spec = pltpu.PrefetchScalarGridSpec(
        num_scalar_prefetch=1,
        grid=(grid,),
        in_specs=[
            pl.BlockSpec(memory_space=pltpu.HBM),
            pl.BlockSpec((EMB_DIM, N_CLASSES), lambda i, xs: (0, 0)),
            pl.BlockSpec((1, N_CLASSES), lambda i, xs: (0, 0)),
        ],
        out_specs=pl.BlockSpec((rows, N_CLASSES), lambda i, xs: (i, 0)),
        scratch_shapes=[
            pltpu.VMEM((rows, EMB_DIM), jnp.float32),
            pltpu.SemaphoreType.DMA,
        ],
    )
    return pl.pallas_call(
        body,
        grid_spec=grid_spec,
        out_shape=jax.ShapeDtypeStruct((BATCH, N_CLASSES), jnp.float32),
    )(idx, emb, wt, b2)


@jax.jit
def kernel(x, emb, W, b):
    idx = x.astype(jnp.int32)
    return _tc_fused(idx, emb, W.T, b.reshape(1, N_CLASSES))


# speedup vs baseline: 1.4138x; 1.4138x over previous
import functools

import jax
import jax.numpy as jnp
from jax import lax
from jax.experimental import pallas as pl
from jax.experimental.pallas import tpu as pltpu
from jax.experimental.pallas import tpu_sc as plsc

BATCH = 16384
VOCAB = 1000000
EMB_DIM = 2
N_CLASSES = 100

_NC = 2
_NS = 16
_NW = _NC * _NS
_PW = BATCH // _NW
_NG = _PW // 16


def _sc_gather(table, idx):
    mesh = plsc.VectorSubcoreMesh(core_axis_name="c", subcore_axis_name="s")

    @functools.partial(
        pl.kernel,
        out_type=jax.ShapeDtypeStruct((EMB_DIM, BATCH), jnp.float32),
        mesh=mesh,
        scratch_types=[
            pltpu.VMEM((_PW,), jnp.int32),
            pltpu.VMEM((_PW, EMB_DIM), jnp.float32),
            pltpu.VMEM((_PW,), jnp.float32),
            pltpu.VMEM((_PW,), jnp.float32),
            pltpu.SemaphoreType.DMA,
        ],
        compiler_params=pltpu.CompilerParams(
            use_tc_tiling_on_sc=True, needs_layout_passes=False
        ),
    )
    def gather_kernel(tbl, idx_h, out_h, idx_v, buf_v, e0_v, e1_v, sem):
        wid = lax.axis_index("s") * _NC + lax.axis_index("c")
        base = wid * _PW
        pltpu.sync_copy(idx_h.at[wid], idx_v)
        lanes = lax.iota(jnp.int32, 16)

        def fire_group(m):
            v = idx_v[pl.ds(m * 16, 16)]
            for t in range(16):
                i0 = lax.reduce_sum(jnp.where(lanes == t, v, 0), axes=(0,))
                pltpu.async_copy(
                    tbl.at[pl.ds(i0, 1)], buf_v.at[pl.ds(m * 16 + t, 1)], sem
                )

        def drain_group(m):
            for t in range(16):
                pltpu.make_async_copy(
                    tbl.at[pl.ds(0, 1)], buf_v.at[pl.ds(m * 16 + t, 1)], sem
                ).wait()

        def body(m, carry):
            fire_group(m)

            @pl.when(m > 0)
            def _():
                drain_group(m - 1)

            return carry

        lax.fori_loop(0, _NG, body, 0)
        drain_group(_NG - 1)

        zeros = jnp.zeros((16,), jnp.int32)
        ones = zeros + 1
        for m in range(_NG):
            rid = lanes + (m * 16)
            e0_v[pl.ds(m * 16, 16)] = plsc.load_gather(buf_v, [rid, zeros])
            e1_v[pl.ds(m * 16, 16)] = plsc.load_gather(buf_v, [rid, ones])
        pltpu.sync_copy(e0_v, out_h.at[0, pl.ds(base, _PW)])
        pltpu.sync_copy(e1_v, out_h.at[1, pl.ds(base, _PW)])

    return gather_kernel(table, idx)


def _tc_dense_softmax(eT, wt, b2):
    rows = 2048
    grid = BATCH // rows

    def body(et_ref, wt_ref, b_ref, out_ref):
        logits = lax.dot_general(
            et_ref[...], wt_ref[...],
            (((0,), (0,)), ((), ())),
            preferred_element_type=jnp.float32,
        ) + b_ref[...]
        m = jnp.max(logits, axis=1, keepdims=True)
        p = jnp.exp(logits - m)
        out_ref[...] = p / jnp.sum(p, axis=1, keepdims=True)

    return pl.pallas_call(
        body,
        grid=(grid,),
        in_specs=[
            pl.BlockSpec((EMB_DIM, rows), lambda i: (0, i)),
            pl.BlockSpec((EMB_DIM, N_CLASSES), lambda i: (0, 0)),
            pl.BlockSpec((1, N_CLASSES), lambda i: (0, 0)),
        ],
        out_specs=pl.BlockSpec((rows, N_CLASSES), lambda i: (i, 0)),
        out_shape=jax.ShapeDtypeStruct((BATCH, N_CLASSES), jnp.float32),
    )(eT, wt, b2)


def _tc_fused(idx, emb, wt, b2):
    rows = 2048
    grid = BATCH // rows

    unroll = 4
    lag = 256

    def body(xs_ref, emb_ref, wt_ref, b_ref, out_ref, buf, sem):
        step = pl.program_id(0)
        base = step * rows

        def start_row(k):
            ix = xs_ref[base + k]
            pltpu.make_async_copy(
                emb_ref.at[pl.ds(ix, 1)], buf.at[pl.ds(k, 1)], sem
            ).start()

        def wait_row(k):
            pltpu.make_async_copy(
                emb_ref.at[pl.ds(0, 1)], buf.at[pl.ds(k, 1)], sem
            ).wait()

        def issue(m, carry):
            for u in range(unroll):
                start_row(m * unroll + u)

            @pl.when(m >= lag // unroll)
            def _():
                for u in range(unroll):
                    wait_row((m - lag // unroll) * unroll + u)

            return carry

        lax.fori_loop(0, rows // unroll, issue, 0)

        def drain(m, carry):
            for u in range(unroll):
                wait_row(m * unroll + u)
            return carry

        lax.fori_loop((rows - lag) // unroll, rows // unroll, drain, 0)

        logits = jnp.dot(
            buf[...], wt_ref[...], preferred_element_type=jnp.float32
        ) + b_ref[...]
        m = jnp.max(logits, axis=1, keepdims=True)
        p = jnp.exp(logits - m)
        out_ref[...] = p / jnp.sum(p, axis=1, keepdims=True)

    grid_spec = pltpu.PrefetchScalarGridSpec(
        num_scalar_prefetch=1,
        grid=(grid,),
        in_specs=[
            pl.BlockSpec(memory_space=pltpu.HBM),
            pl.BlockSpec((EMB_DIM, N_CLASSES), lambda i, xs: (0, 0)),
            pl.BlockSpec((1, N_CLASSES), lambda i, xs: (0, 0)),
        ],
        out_specs=pl.BlockSpec((rows, N_CLASSES), lambda i, xs: (i, 0)),
        scratch_shapes=[
            pltpu.VMEM((rows, EMB_DIM), jnp.float32),
            pltpu.SemaphoreType.DMA,
        ],
    )
    return pl.pallas_call(
        body,
        grid_spec=grid_spec,
        out_shape=jax.ShapeDtypeStruct((BATCH, N_CLASSES), jnp.float32),
    )(idx, emb, wt, b2)


@jax.jit
def kernel(x, emb, W, b):
    idx = x.astype(jnp.int32)
    return _tc_fused(idx, emb, W.T, b.reshape(1, N_CLASSES))


# speedup vs baseline: 1.4289x; 1.0106x over previous
import functools

import jax
import jax.numpy as jnp
from jax import lax
from jax.experimental import pallas as pl
from jax.experimental.pallas import tpu as pltpu
from jax.experimental.pallas import tpu_sc as plsc

BATCH = 16384
VOCAB = 1000000
EMB_DIM = 2
N_CLASSES = 100

_NC = 2
_NS = 16
_NW = _NC * _NS
_PW = BATCH // _NW
_NG = _PW // 16


def _sc_gather(table, idx):
    mesh = plsc.VectorSubcoreMesh(core_axis_name="c", subcore_axis_name="s")

    @functools.partial(
        pl.kernel,
        out_type=jax.ShapeDtypeStruct((EMB_DIM, BATCH), jnp.float32),
        mesh=mesh,
        scratch_types=[
            pltpu.VMEM((_PW,), jnp.int32),
            pltpu.VMEM((_PW, EMB_DIM), jnp.float32),
            pltpu.VMEM((_PW,), jnp.float32),
            pltpu.VMEM((_PW,), jnp.float32),
            pltpu.SemaphoreType.DMA,
        ],
        compiler_params=pltpu.CompilerParams(
            use_tc_tiling_on_sc=True, needs_layout_passes=False
        ),
    )
    def gather_kernel(tbl, idx_h, out_h, idx_v, buf_v, e0_v, e1_v, sem):
        wid = lax.axis_index("s") * _NC + lax.axis_index("c")
        base = wid * _PW
        pltpu.sync_copy(idx_h.at[wid], idx_v)
        lanes = lax.iota(jnp.int32, 16)

        def fire_group(m):
            v = idx_v[pl.ds(m * 16, 16)]
            for t in range(16):
                i0 = lax.reduce_sum(jnp.where(lanes == t, v, 0), axes=(0,))
                pltpu.async_copy(
                    tbl.at[pl.ds(i0, 1)], buf_v.at[pl.ds(m * 16 + t, 1)], sem
                )

        def drain_group(m):
            for t in range(16):
                pltpu.make_async_copy(
                    tbl.at[pl.ds(0, 1)], buf_v.at[pl.ds(m * 16 + t, 1)], sem
                ).wait()

        def body(m, carry):
            fire_group(m)

            @pl.when(m > 0)
            def _():
                drain_group(m - 1)

            return carry

        lax.fori_loop(0, _NG, body, 0)
        drain_group(_NG - 1)

        zeros = jnp.zeros((16,), jnp.int32)
        ones = zeros + 1
        for m in range(_NG):
            rid = lanes + (m * 16)
            e0_v[pl.ds(m * 16, 16)] = plsc.load_gather(buf_v, [rid, zeros])
            e1_v[pl.ds(m * 16, 16)] = plsc.load_gather(buf_v, [rid, ones])
        pltpu.sync_copy(e0_v, out_h.at[0, pl.ds(base, _PW)])
        pltpu.sync_copy(e1_v, out_h.at[1, pl.ds(base, _PW)])

    return gather_kernel(table, idx)


def _tc_dense_softmax(eT, wt, b2):
    rows = 2048
    grid = BATCH // rows

    def body(et_ref, wt_ref, b_ref, out_ref):
        logits = lax.dot_general(
            et_ref[...], wt_ref[...],
            (((0,), (0,)), ((), ())),
            preferred_element_type=jnp.float32,
        ) + b_ref[...]
        m = jnp.max(logits, axis=1, keepdims=True)
        p = jnp.exp(logits - m)
        out_ref[...] = p / jnp.sum(p, axis=1, keepdims=True)

    return pl.pallas_call(
        body,
        grid=(grid,),
        in_specs=[
            pl.BlockSpec((EMB_DIM, rows), lambda i: (0, i)),
            pl.BlockSpec((EMB_DIM, N_CLASSES), lambda i: (0, 0)),
            pl.BlockSpec((1, N_CLASSES), lambda i: (0, 0)),
        ],
        out_specs=pl.BlockSpec((rows, N_CLASSES), lambda i: (i, 0)),
        out_shape=jax.ShapeDtypeStruct((BATCH, N_CLASSES), jnp.float32),
    )(eT, wt, b2)


def _tc_fused(idx, emb, wt, b2):
    rows = 2048
    grid = BATCH // rows

    unroll = 8
    lag = 256

    def body(xs_ref, emb_ref, wt_ref, b_ref, out_ref, buf, sem):
        step = pl.program_id(0)
        base = step * rows

        def start_row(k):
            ix = xs_ref[base + k]
            pltpu.make_async_copy(
                emb_ref.at[pl.ds(ix, 1)], buf.at[pl.ds(k, 1)], sem
            ).start()

        def wait_group(k0):
            pltpu.make_async_copy(
                emb_ref.at[pl.ds(0, unroll)], buf.at[pl.ds(k0, unroll)], sem
            ).wait()

        def issue(m, carry):
            for u in range(unroll):
                start_row(m * unroll + u)

            @pl.when(m >= lag // unroll)
            def _():
                wait_group((m - lag // unroll) * unroll)

            return carry

        lax.fori_loop(0, rows // unroll, issue, 0)

        def drain(m, carry):
            wait_group(m * unroll)
            return carry

        lax.fori_loop((rows - lag) // unroll, rows // unroll, drain, 0)

        logits = jnp.dot(
            buf[...], wt_ref[...], preferred_element_type=jnp.float32
        ) + b_ref[...]
        m = jnp.max(logits, axis=1, keepdims=True)
        p = jnp.exp(logits - m)
        out_ref[...] = p / jnp.sum(p, axis=1, keepdims=True)

    grid_spec = pltpu.PrefetchScalarGridSpec(
        num_scalar_prefetch=1,
        grid=(grid,),
        in_specs=[
            pl.BlockSpec(memory_space=pltpu.HBM),
            pl.BlockSpec((EMB_DIM, N_CLASSES), lambda i, xs: (0, 0)),
            pl.BlockSpec((1, N_CLASSES), lambda i, xs: (0, 0)),
        ],
        out_specs=pl.BlockSpec((rows, N_CLASSES), lambda i, xs: (i, 0)),
        scratch_shapes=[
            pltpu.VMEM((rows, EMB_DIM), jnp.float32),
            pltpu.SemaphoreType.DMA,
        ],
    )
    return pl.pallas_call(
        body,
        grid_spec=grid_spec,
        out_shape=jax.ShapeDtypeStruct((BATCH, N_CLASSES), jnp.float32),
    )(idx, emb, wt, b2)


@jax.jit
def kernel(x, emb, W, b):
    idx = x.astype(jnp.int32)
    return _tc_fused(idx, emb, W.T, b.reshape(1, N_CLASSES))


# speedup vs baseline: 1.4290x; 1.0001x over previous
import functools

import jax
import jax.numpy as jnp
from jax import lax
from jax.experimental import pallas as pl
from jax.experimental.pallas import tpu as pltpu
from jax.experimental.pallas import tpu_sc as plsc

BATCH = 16384
VOCAB = 1000000
EMB_DIM = 2
N_CLASSES = 100

_NC = 2
_NS = 16
_NW = _NC * _NS
_PW = BATCH // _NW
_NG = _PW // 16


def _sc_gather(table, idx):
    mesh = plsc.VectorSubcoreMesh(core_axis_name="c", subcore_axis_name="s")

    @functools.partial(
        pl.kernel,
        out_type=jax.ShapeDtypeStruct((EMB_DIM, BATCH), jnp.float32),
        mesh=mesh,
        scratch_types=[
            pltpu.VMEM((_PW,), jnp.int32),
            pltpu.VMEM((_PW, EMB_DIM), jnp.float32),
            pltpu.VMEM((_PW,), jnp.float32),
            pltpu.VMEM((_PW,), jnp.float32),
            pltpu.SemaphoreType.DMA,
        ],
        compiler_params=pltpu.CompilerParams(
            use_tc_tiling_on_sc=True, needs_layout_passes=False
        ),
    )
    def gather_kernel(tbl, idx_h, out_h, idx_v, buf_v, e0_v, e1_v, sem):
        wid = lax.axis_index("s") * _NC + lax.axis_index("c")
        base = wid * _PW
        pltpu.sync_copy(idx_h.at[wid], idx_v)
        lanes = lax.iota(jnp.int32, 16)

        def fire_group(m):
            v = idx_v[pl.ds(m * 16, 16)]
            for t in range(16):
                i0 = lax.reduce_sum(jnp.where(lanes == t, v, 0), axes=(0,))
                pltpu.async_copy(
                    tbl.at[pl.ds(i0, 1)], buf_v.at[pl.ds(m * 16 + t, 1)], sem
                )

        def drain_group(m):
            for t in range(16):
                pltpu.make_async_copy(
                    tbl.at[pl.ds(0, 1)], buf_v.at[pl.ds(m * 16 + t, 1)], sem
                ).wait()

        def body(m, carry):
            fire_group(m)

            @pl.when(m > 0)
            def _():
                drain_group(m - 1)

            return carry

        lax.fori_loop(0, _NG, body, 0)
        drain_group(_NG - 1)

        zeros = jnp.zeros((16,), jnp.int32)
        ones = zeros + 1
        for m in range(_NG):
            rid = lanes + (m * 16)
            e0_v[pl.ds(m * 16, 16)] = plsc.load_gather(buf_v, [rid, zeros])
            e1_v[pl.ds(m * 16, 16)] = plsc.load_gather(buf_v, [rid, ones])
        pltpu.sync_copy(e0_v, out_h.at[0, pl.ds(base, _PW)])
        pltpu.sync_copy(e1_v, out_h.at[1, pl.ds(base, _PW)])

    return gather_kernel(table, idx)


def _tc_dense_softmax(eT, wt, b2):
    rows = 2048
    grid = BATCH // rows

    def body(et_ref, wt_ref, b_ref, out_ref):
        logits = lax.dot_general(
            et_ref[...], wt_ref[...],
            (((0,), (0,)), ((), ())),
            preferred_element_type=jnp.float32,
        ) + b_ref[...]
        m = jnp.max(logits, axis=1, keepdims=True)
        p = jnp.exp(logits - m)
        out_ref[...] = p / jnp.sum(p, axis=1, keepdims=True)

    return pl.pallas_call(
        body,
        grid=(grid,),
        in_specs=[
            pl.BlockSpec((EMB_DIM, rows), lambda i: (0, i)),
            pl.BlockSpec((EMB_DIM, N_CLASSES), lambda i: (0, 0)),
            pl.BlockSpec((1, N_CLASSES), lambda i: (0, 0)),
        ],
        out_specs=pl.BlockSpec((rows, N_CLASSES), lambda i: (i, 0)),
        out_shape=jax.ShapeDtypeStruct((BATCH, N_CLASSES), jnp.float32),
    )(eT, wt, b2)


def _tc_fused(idx, emb, wt, b2):
    rows = 2048
    grid = BATCH // rows

    unroll = 16
    nsem = 4
    lag = 512

    def body(xs_ref, emb_ref, wt_ref, b_ref, out_ref, buf, *sems):
        step = pl.program_id(0)
        base = step * rows

        def start_row(k, u):
            ix = xs_ref[base + k]
            pltpu.make_async_copy(
                emb_ref.at[pl.ds(ix, 1)], buf.at[pl.ds(k, 1)], sems[u % nsem]
            ).start()

        def wait_group(k0):
            share = unroll // nsem
            for s in range(nsem):
                pltpu.make_async_copy(
                    emb_ref.at[pl.ds(0, share)],
                    buf.at[pl.ds(k0 + s * share, share)],
                    sems[s],
                ).wait()

        def issue(m, carry):
            for u in range(unroll):
                start_row(m * unroll + u, u)

            @pl.when(m >= lag // unroll)
            def _():
                wait_group((m - lag // unroll) * unroll)

            return carry

        lax.fori_loop(0, rows // unroll, issue, 0)

        def drain(m, carry):
            wait_group(m * unroll)
            return carry

        lax.fori_loop((rows - lag) // unroll, rows // unroll, drain, 0)

        logits = jnp.dot(
            buf[...], wt_ref[...], preferred_element_type=jnp.float32
        ) + b_ref[...]
        m = jnp.max(logits, axis=1, keepdims=True)
        p = jnp.exp(logits - m)
        out_ref[...] = p / jnp.sum(p, axis=1, keepdims=True)

    grid_spec = pltpu.PrefetchScalarGridSpec(
        num_scalar_prefetch=1,
        grid=(grid,),
        in_specs=[
            pl.BlockSpec(memory_space=pltpu.HBM),
            pl.BlockSpec((EMB_DIM, N_CLASSES), lambda i, xs: (0, 0)),
            pl.BlockSpec((1, N_CLASSES), lambda i, xs: (0, 0)),
        ],
        out_specs=pl.BlockSpec((rows, N_CLASSES), lambda i, xs: (i, 0)),
        scratch_shapes=(
            [pltpu.VMEM((rows, EMB_DIM), jnp.float32)]
            + [pltpu.SemaphoreType.DMA] * 4
        ),
    )
    return pl.pallas_call(
        body,
        grid_spec=grid_spec,
        out_shape=jax.ShapeDtypeStruct((BATCH, N_CLASSES), jnp.float32),
    )(idx, emb, wt, b2)


@jax.jit
def kernel(x, emb, W, b):
    idx = x.astype(jnp.int32)
    return _tc_fused(idx, emb, W.T, b.reshape(1, N_CLASSES))


# speedup vs baseline: 1.6389x; 1.1469x over previous
import functools

import jax
import jax.numpy as jnp
from jax import lax
from jax.experimental import pallas as pl
from jax.experimental.pallas import tpu as pltpu
from jax.experimental.pallas import tpu_sc as plsc

BATCH = 16384
VOCAB = 1000000
EMB_DIM = 2
N_CLASSES = 100

_NC = 2
_NS = 16
_NW = _NC * _NS
_PW = BATCH // _NW
_NG = _PW // 16


def _sc_gather(table, idx):
    mesh = plsc.VectorSubcoreMesh(core_axis_name="c", subcore_axis_name="s")

    @functools.partial(
        pl.kernel,
        out_type=jax.ShapeDtypeStruct((EMB_DIM, BATCH), jnp.float32),
        mesh=mesh,
        scratch_types=[
            pltpu.VMEM((_PW,), jnp.int32),
            pltpu.VMEM((_PW, EMB_DIM), jnp.float32),
            pltpu.VMEM((_PW,), jnp.float32),
            pltpu.VMEM((_PW,), jnp.float32),
            pltpu.SemaphoreType.DMA,
        ],
        compiler_params=pltpu.CompilerParams(
            use_tc_tiling_on_sc=True, needs_layout_passes=False
        ),
    )
    def gather_kernel(tbl, idx_h, out_h, idx_v, buf_v, e0_v, e1_v, sem):
        wid = lax.axis_index("s") * _NC + lax.axis_index("c")
        base = wid * _PW
        pltpu.sync_copy(idx_h.at[wid], idx_v)
        lanes = lax.iota(jnp.int32, 16)

        def fire_group(m):
            v = idx_v[pl.ds(m * 16, 16)]
            for t in range(16):
                i0 = lax.reduce_sum(jnp.where(lanes == t, v, 0), axes=(0,))
                pltpu.async_copy(
                    tbl.at[pl.ds(i0, 1)], buf_v.at[pl.ds(m * 16 + t, 1)], sem
                )

        def drain_group(m):
            for t in range(16):
                pltpu.make_async_copy(
                    tbl.at[pl.ds(0, 1)], buf_v.at[pl.ds(m * 16 + t, 1)], sem
                ).wait()

        def body(m, carry):
            fire_group(m)

            @pl.when(m > 0)
            def _():
                drain_group(m - 1)

            return carry

        lax.fori_loop(0, _NG, body, 0)
        drain_group(_NG - 1)

        zeros = jnp.zeros((16,), jnp.int32)
        ones = zeros + 1
        for m in range(_NG):
            rid = lanes + (m * 16)
            e0_v[pl.ds(m * 16, 16)] = plsc.load_gather(buf_v, [rid, zeros])
            e1_v[pl.ds(m * 16, 16)] = plsc.load_gather(buf_v, [rid, ones])
        pltpu.sync_copy(e0_v, out_h.at[0, pl.ds(base, _PW)])
        pltpu.sync_copy(e1_v, out_h.at[1, pl.ds(base, _PW)])

    return gather_kernel(table, idx)


def _tc_dense_softmax(eT, wt, b2):
    rows = 2048
    grid = BATCH // rows

    def body(et_ref, wt_ref, b_ref, out_ref):
        logits = lax.dot_general(
            et_ref[...], wt_ref[...],
            (((0,), (0,)), ((), ())),
            preferred_element_type=jnp.float32,
        ) + b_ref[...]
        m = jnp.max(logits, axis=1, keepdims=True)
        p = jnp.exp(logits - m)
        out_ref[...] = p / jnp.sum(p, axis=1, keepdims=True)

    return pl.pallas_call(
        body,
        grid=(grid,),
        in_specs=[
            pl.BlockSpec((EMB_DIM, rows), lambda i: (0, i)),
            pl.BlockSpec((EMB_DIM, N_CLASSES), lambda i: (0, 0)),
            pl.BlockSpec((1, N_CLASSES), lambda i: (0, 0)),
        ],
        out_specs=pl.BlockSpec((rows, N_CLASSES), lambda i: (i, 0)),
        out_shape=jax.ShapeDtypeStruct((BATCH, N_CLASSES), jnp.float32),
    )(eT, wt, b2)


@jax.jit
def kernel(x, emb, W, b):
    idx = x.astype(jnp.int32).reshape(_NW, _PW)
    eT = _sc_gather(emb, idx)
    return _tc_dense_softmax(eT, W.T, b.reshape(1, N_CLASSES))


# speedup vs baseline: 1.6557x; 1.0103x over previous
import functools

import jax
import jax.numpy as jnp
from jax import lax
from jax.experimental import pallas as pl
from jax.experimental.pallas import tpu as pltpu
from jax.experimental.pallas import tpu_sc as plsc

BATCH = 16384
VOCAB = 1000000
EMB_DIM = 2
N_CLASSES = 100

_NC = 2
_NS = 16
_NW = _NC * _NS
_PW = BATCH // _NW
_NG = _PW // 16


def _sc_gather(table, idx):
    mesh = plsc.VectorSubcoreMesh(core_axis_name="c", subcore_axis_name="s")

    @functools.partial(
        pl.kernel,
        out_type=jax.ShapeDtypeStruct((EMB_DIM, BATCH), jnp.float32),
        mesh=mesh,
        scratch_types=[
            pltpu.VMEM((_PW,), jnp.int32),
            pltpu.VMEM((_PW, EMB_DIM), jnp.float32),
            pltpu.VMEM((_PW,), jnp.float32),
            pltpu.VMEM((_PW,), jnp.float32),
            pltpu.SemaphoreType.DMA,
        ],
        compiler_params=pltpu.CompilerParams(
            use_tc_tiling_on_sc=True, needs_layout_passes=False
        ),
    )
    def gather_kernel(tbl, idx_h, out_h, idx_v, buf_v, e0_v, e1_v, sem):
        wid = lax.axis_index("s") * _NC + lax.axis_index("c")
        base = wid * _PW
        pltpu.sync_copy(idx_h.at[wid], idx_v)
        lanes = lax.iota(jnp.int32, 16)

        def fire_group(m):
            v = idx_v[pl.ds(m * 16, 16)]
            for t in range(16):
                i0 = lax.reduce_sum(jnp.where(lanes == t, v, 0), axes=(0,))
                pltpu.async_copy(
                    tbl.at[pl.ds(i0, 1)], buf_v.at[pl.ds(m * 16 + t, 1)], sem
                )

        def drain_group(m):
            for t in range(16):
                pltpu.make_async_copy(
                    tbl.at[pl.ds(0, 1)], buf_v.at[pl.ds(m * 16 + t, 1)], sem
                ).wait()

        def body(m, carry):
            fire_group(m)

            @pl.when(m > 1)
            def _():
                drain_group(m - 2)

            return carry

        lax.fori_loop(0, _NG, body, 0)
        drain_group(_NG - 2)
        drain_group(_NG - 1)

        zeros = jnp.zeros((16,), jnp.int32)
        ones = zeros + 1
        for m in range(_NG):
            rid = lanes + (m * 16)
            e0_v[pl.ds(m * 16, 16)] = plsc.load_gather(buf_v, [rid, zeros])
            e1_v[pl.ds(m * 16, 16)] = plsc.load_gather(buf_v, [rid, ones])
        pltpu.sync_copy(e0_v, out_h.at[0, pl.ds(base, _PW)])
        pltpu.sync_copy(e1_v, out_h.at[1, pl.ds(base, _PW)])

    return gather_kernel(table, idx)


def _tc_dense_softmax(eT, wt, b2):
    rows = 2048
    grid = BATCH // rows

    def body(et_ref, wt_ref, b_ref, out_ref):
        logits = lax.dot_general(
            et_ref[...], wt_ref[...],
            (((0,), (0,)), ((), ())),
            preferred_element_type=jnp.float32,
        ) + b_ref[...]
        m = jnp.max(logits, axis=1, keepdims=True)
        p = jnp.exp(logits - m)
        out_ref[...] = p / jnp.sum(p, axis=1, keepdims=True)

    return pl.pallas_call(
        body,
        grid=(grid,),
        in_specs=[
            pl.BlockSpec((EMB_DIM, rows), lambda i: (0, i)),
            pl.BlockSpec((EMB_DIM, N_CLASSES), lambda i: (0, 0)),
            pl.BlockSpec((1, N_CLASSES), lambda i: (0, 0)),
        ],
        out_specs=pl.BlockSpec((rows, N_CLASSES), lambda i: (i, 0)),
        out_shape=jax.ShapeDtypeStruct((BATCH, N_CLASSES), jnp.float32),
    )(eT, wt, b2)


@jax.jit
def kernel(x, emb, W, b):
    idx = x.astype(jnp.int32).reshape(_NW, _PW)
    eT = _sc_gather(emb, idx)
    return _tc_dense_softmax(eT, W.T, b.reshape(1, N_CLASSES))


# speedup vs baseline: 1.6621x; 1.0039x over previous
import functools

import jax
import jax.numpy as jnp
from jax import lax
from jax.experimental import pallas as pl
from jax.experimental.pallas import tpu as pltpu
from jax.experimental.pallas import tpu_sc as plsc

BATCH = 16384
VOCAB = 1000000
EMB_DIM = 2
N_CLASSES = 100

_NC = 2
_NS = 16
_NW = _NC * _NS
_PW = BATCH // _NW
_NG = _PW // 16


def _sc_gather(table, idx):
    mesh = plsc.VectorSubcoreMesh(core_axis_name="c", subcore_axis_name="s")

    @functools.partial(
        pl.kernel,
        out_type=jax.ShapeDtypeStruct((EMB_DIM, BATCH), jnp.float32),
        mesh=mesh,
        scratch_types=[
            pltpu.VMEM((_PW,), jnp.int32),
            pltpu.VMEM((_PW, EMB_DIM), jnp.float32),
            pltpu.VMEM((_PW,), jnp.float32),
            pltpu.VMEM((_PW,), jnp.float32),
            pltpu.SemaphoreType.DMA,
        ],
        compiler_params=pltpu.CompilerParams(
            use_tc_tiling_on_sc=True, needs_layout_passes=False
        ),
    )
    def gather_kernel(tbl, idx_h, out_h, idx_v, buf_v, e0_v, e1_v, sem):
        wid = lax.axis_index("s") * _NC + lax.axis_index("c")
        base = wid * _PW
        pltpu.sync_copy(idx_h.at[wid], idx_v)
        lanes = lax.iota(jnp.int32, 16)

        def fire_group(m):
            v = idx_v[pl.ds(m * 16, 16)]
            for t in range(16):
                i0 = lax.reduce_sum(jnp.where(lanes == t, v, 0), axes=(0,))
                pltpu.async_copy(
                    tbl.at[pl.ds(i0, 1)], buf_v.at[pl.ds(m * 16 + t, 1)], sem
                )

        def drain_group(m):
            for t in range(16):
                pltpu.make_async_copy(
                    tbl.at[pl.ds(0, 1)], buf_v.at[pl.ds(m * 16 + t, 1)], sem
                ).wait()

        zeros = jnp.zeros((16,), jnp.int32)
        ones = zeros + 1

        def extract_group(m):
            rid = lanes + m * 16
            e0_v[pl.ds(m * 16, 16)] = plsc.load_gather(buf_v, [rid, zeros])
            e1_v[pl.ds(m * 16, 16)] = plsc.load_gather(buf_v, [rid, ones])

        def body(m, carry):
            fire_group(m)

            @pl.when(m > 1)
            def _():
                drain_group(m - 2)
                extract_group(m - 2)

            return carry

        lax.fori_loop(0, _NG, body, 0)
        for g in (_NG - 2, _NG - 1):
            drain_group(g)
            extract_group(g)
        pltpu.sync_copy(e0_v, out_h.at[0, pl.ds(base, _PW)])
        pltpu.sync_copy(e1_v, out_h.at[1, pl.ds(base, _PW)])

    return gather_kernel(table, idx)


def _tc_dense_softmax(eT, wt, b2):
    rows = 2048
    grid = BATCH // rows

    def body(et_ref, wt_ref, b_ref, out_ref):
        logits = lax.dot_general(
            et_ref[...], wt_ref[...],
            (((0,), (0,)), ((), ())),
            preferred_element_type=jnp.float32,
        ) + b_ref[...]
        m = jnp.max(logits, axis=1, keepdims=True)
        p = jnp.exp(logits - m)
        out_ref[...] = p / jnp.sum(p, axis=1, keepdims=True)

    return pl.pallas_call(
        body,
        grid=(grid,),
        in_specs=[
            pl.BlockSpec((EMB_DIM, rows), lambda i: (0, i)),
            pl.BlockSpec((EMB_DIM, N_CLASSES), lambda i: (0, 0)),
            pl.BlockSpec((1, N_CLASSES), lambda i: (0, 0)),
        ],
        out_specs=pl.BlockSpec((rows, N_CLASSES), lambda i: (i, 0)),
        out_shape=jax.ShapeDtypeStruct((BATCH, N_CLASSES), jnp.float32),
    )(eT, wt, b2)


@jax.jit
def kernel(x, emb, W, b):
    idx = x.astype(jnp.int32).reshape(_NW, _PW)
    eT = _sc_gather(emb, idx)
    return _tc_dense_softmax(eT, W.T, b.reshape(1, N_CLASSES))
